# baseline (speedup 1.0000x reference)
import functools

import jax
import jax.numpy as jnp
from jax import lax
from jax.experimental import pallas as pl
from jax.experimental.pallas import tpu as pltpu
from jax.experimental.pallas import tpu_sc as plsc

B = 16384
D = 64
W = 130

_NC = 2
_NS = 16
_NW = _NC * _NS
_CPW = B // _NW
_GCH = 128
_NG = _CPW // _GCH


def _body(scale_hbm, gender_hbm, age_hbm, perp_hbm, gtab_hbm, ptab_hbm,
          out_hbm, pidx_v, gidx_v, pbuf_v, gbuf_v, sa_v, obuf_v, sems):
    wid = lax.axis_index("s") * _NC + lax.axis_index("c")
    base = wid * _CPW

    pltpu.sync_copy(perp_hbm.at[wid], pidx_v)
    pltpu.sync_copy(gender_hbm.at[wid], gidx_v)
    pltpu.sync_copy(scale_hbm.at[wid], sa_v.at[0])
    pltpu.sync_copy(age_hbm.at[wid], sa_v.at[1])

    def gather(j, slot):
        pltpu.make_async_copy(
            ptab_hbm.at[pidx_v.at[j]], pbuf_v.at[slot], sems.at[slot]).start()
        pltpu.make_async_copy(
            gtab_hbm.at[gidx_v.at[j]], gbuf_v.at[slot], sems.at[slot]).start()

    def wait(j, slot):
        pltpu.make_async_copy(
            ptab_hbm.at[pidx_v.at[j]], pbuf_v.at[slot], sems.at[slot]).wait()
        pltpu.make_async_copy(
            gtab_hbm.at[gidx_v.at[j]], gbuf_v.at[slot], sems.at[slot]).wait()

    lanes = lax.iota(jnp.int32, 16)
    goffs = [1 + 16 * k + lanes for k in range(D // 16)]
    poffs = [66 + 16 * k + lanes for k in range(D // 16)]

    gather(0, 0)
    gather(1, 1)
    for j in range(_NG):
        slot = j % 2
        wait(j, slot)

        def row_body(r, carry):
            rb = (j * _GCH + r) * W
            for k in range(D // 16):
                plsc.store_scatter(obuf_v, [rb + goffs[k]],
                                   gbuf_v[slot, r, pl.ds(16 * k, 16)])
                plsc.store_scatter(obuf_v, [rb + poffs[k]],
                                   pbuf_v[slot, r, pl.ds(16 * k, 16)])
            return carry

        lax.fori_loop(0, _GCH, row_body, 0, unroll=4)
        if j + 2 < _NG:
            gather(j + 2, slot)

    def sa_body(i, carry):
        rvec = (16 * i + lanes) * W
        plsc.store_scatter(obuf_v, [rvec], sa_v[0, pl.ds(16 * i, 16)])
        plsc.store_scatter(obuf_v, [rvec + 65], sa_v[1, pl.ds(16 * i, 16)])
        return carry

    lax.fori_loop(0, _CPW // 16, sa_body, 0, unroll=4)

    pltpu.sync_copy(obuf_v, out_hbm.at[pl.ds(base * W, _CPW * W)])


@jax.jit
def kernel(scale, gender, age, perp, gender_table, perp_table):
    mesh = plsc.VectorSubcoreMesh(core_axis_name="c", subcore_axis_name="s")
    run = pl.kernel(
        _body,
        mesh=mesh,
        compiler_params=pltpu.CompilerParams(
            use_tc_tiling_on_sc=False, needs_layout_passes=False),
        out_type=jax.ShapeDtypeStruct((B * W,), jnp.float32),
        scratch_types=[
            pltpu.VMEM((_NG, _GCH), jnp.int32),
            pltpu.VMEM((_NG, _GCH), jnp.int32),
            pltpu.VMEM((2, _GCH, D), jnp.float32),
            pltpu.VMEM((2, _GCH, D), jnp.float32),
            pltpu.VMEM((2, _CPW), jnp.float32),
            pltpu.VMEM((_CPW * W,), jnp.float32),
            pltpu.SemaphoreType.DMA((2,)),
        ],
    )
    flat = run(scale.reshape(_NW, _CPW), gender.reshape(_NW, _NG, _GCH),
               age.reshape(_NW, _CPW), perp.reshape(_NW, _NG, _GCH),
               gender_table, perp_table)
    return flat.reshape(B, W)


# speedup vs baseline: 3.3825x; 3.3825x over previous
import jax
import jax.numpy as jnp
from jax import lax
from jax.experimental import pallas as pl
from jax.experimental.pallas import tpu as pltpu
from jax.experimental.pallas import tpu_sc as plsc

B = 16384
D = 64
W = 130

_NC = 2
_NS = 16
_NW = _NC * _NS
_CPW = B // _NW
_GCH = 128
_NG = _CPW // _GCH


def _body(scale_hbm, genderf_hbm, age_hbm, perp_hbm, gtab_hbm, ptab_hbm,
          out_hbm, pidx_v, gf_v, gtab_v, pbuf_v, sa_v, obuf_v, sems, osem):
    wid = lax.axis_index("s") * _NC + lax.axis_index("c")
    base = wid * _CPW

    pltpu.sync_copy(perp_hbm.at[wid], pidx_v)

    gathers = [
        pltpu.make_async_copy(
            ptab_hbm.at[pidx_v.at[j]], pbuf_v.at[j], sems.at[j])
        for j in range(_NG)
    ]
    for g in gathers:
        g.start()

    stage = [
        pltpu.make_async_copy(genderf_hbm.at[wid], gf_v, osem),
        pltpu.make_async_copy(scale_hbm.at[wid], sa_v.at[0], osem),
        pltpu.make_async_copy(age_hbm.at[wid], sa_v.at[1], osem),
        pltpu.make_async_copy(gtab_hbm, gtab_v, osem),
    ]
    for c in stage:
        c.start()
    for c in stage:
        c.wait()

    lanes = lax.iota(jnp.int32, 16)
    goffs = [1 + 16 * k + lanes for k in range(D // 16)]
    poffs = [66 + 16 * k + lanes for k in range(D // 16)]

    g0 = [gtab_v[0, pl.ds(16 * k, 16)] for k in range(D // 16)]
    g1 = [gtab_v[1, pl.ds(16 * k, 16)] for k in range(D // 16)]

    def sa_body(i, carry):
        rvec = (16 * i + lanes) * W
        plsc.store_scatter(obuf_v, [rvec], sa_v[0, pl.ds(16 * i, 16)])
        plsc.store_scatter(obuf_v, [rvec + 65], sa_v[1, pl.ds(16 * i, 16)])
        return carry

    lax.fori_loop(0, _CPW // 16, sa_body, 0, unroll=4)

    owrites = []
    for j in range(_NG):
        gathers[j].wait()

        def grp_body(i, carry):
            gv = gf_v[pl.ds(j * _GCH + 16 * i, 16)]
            for t in range(16):
                r = 16 * i + t
                rb = (j * _GCH + r) * W
                m = gv[t] != 0.0
                for k in range(D // 16):
                    plsc.store_scatter(obuf_v, [rb + goffs[k]],
                                       jnp.where(m, g1[k], g0[k]))
                    plsc.store_scatter(obuf_v, [rb + poffs[k]],
                                       pbuf_v[j, r, pl.ds(16 * k, 16)])
            return carry

        lax.fori_loop(0, _GCH // 16, grp_body, 0)

        ow = pltpu.make_async_copy(
            obuf_v.at[pl.ds(j * _GCH * W, _GCH * W)],
            out_hbm.at[pl.ds(base * W + j * _GCH * W, _GCH * W)],
            osem)
        ow.start()
        owrites.append(ow)

    for ow in owrites:
        ow.wait()


@jax.jit
def kernel(scale, gender, age, perp, gender_table, perp_table):
    mesh = plsc.VectorSubcoreMesh(core_axis_name="c", subcore_axis_name="s")
    run = pl.kernel(
        _body,
        mesh=mesh,
        compiler_params=pltpu.CompilerParams(
            use_tc_tiling_on_sc=False, needs_layout_passes=False),
        out_type=jax.ShapeDtypeStruct((B * W,), jnp.float32),
        scratch_types=[
            pltpu.VMEM((_NG, _GCH), jnp.int32),
            pltpu.VMEM((_CPW,), jnp.float32),
            pltpu.VMEM((2, D), jnp.float32),
            pltpu.VMEM((_NG, _GCH, D), jnp.float32),
            pltpu.VMEM((2, _CPW), jnp.float32),
            pltpu.VMEM((_CPW * W,), jnp.float32),
            pltpu.SemaphoreType.DMA((_NG,)),
            pltpu.SemaphoreType.DMA,
        ],
    )
    flat = run(scale.reshape(_NW, _CPW),
               gender.reshape(_NW, _CPW).astype(jnp.float32),
               age.reshape(_NW, _CPW), perp.reshape(_NW, _NG, _GCH),
               gender_table, perp_table)
    return flat.reshape(B, W)


# speedup vs baseline: 3.4693x; 1.0257x over previous
import jax
import jax.numpy as jnp
from jax import lax
from jax.experimental import pallas as pl
from jax.experimental.pallas import tpu as pltpu
from jax.experimental.pallas import tpu_sc as plsc

B = 16384
D = 64
W = 130

_NC = 2
_NS = 16
_NW = _NC * _NS
_CPW = B // _NW
_GCH = 128
_NG = _CPW // _GCH


def _body(scale_hbm, genderf_hbm, age_hbm, perp_hbm, gtab_hbm, ptab_hbm,
          out_hbm, pidx_v, gf_v, gtab_v, pbuf_v, sa_v, obuf_v, sems, osem):
    wid = lax.axis_index("s") * _NC + lax.axis_index("c")
    base = wid * _CPW

    pltpu.sync_copy(perp_hbm.at[wid], pidx_v)

    gathers = [
        pltpu.make_async_copy(
            ptab_hbm.at[pidx_v.at[j]], pbuf_v.at[j], sems.at[j])
        for j in range(_NG)
    ]
    for g in gathers:
        g.start()

    stage = [
        pltpu.make_async_copy(genderf_hbm.at[wid], gf_v, osem),
        pltpu.make_async_copy(scale_hbm.at[wid], sa_v.at[0], osem),
        pltpu.make_async_copy(age_hbm.at[wid], sa_v.at[1], osem),
        pltpu.make_async_copy(gtab_hbm, gtab_v, osem),
    ]
    for c in stage:
        c.start()
    for c in stage:
        c.wait()

    lanes = lax.iota(jnp.int32, 16)
    goffs = [1 + 16 * k + lanes for k in range(D // 16)]
    poffs = [66 + 16 * k + lanes for k in range(D // 16)]

    g0 = [gtab_v[0, pl.ds(16 * k, 16)] for k in range(D // 16)]
    g1 = [gtab_v[1, pl.ds(16 * k, 16)] for k in range(D // 16)]

    @plsc.parallel_loop(0, _CPW // 16, 1, unroll=4)
    def _sa(i):
        rvec = (16 * i + lanes) * W
        plsc.store_scatter(obuf_v, [rvec], sa_v[0, pl.ds(16 * i, 16)])
        plsc.store_scatter(obuf_v, [rvec + 65], sa_v[1, pl.ds(16 * i, 16)])

    owrites = []
    for j in range(_NG):
        gathers[j].wait()

        @plsc.parallel_loop(0, _GCH // 16, 1, unroll=2)
        def _grp(i):
            gv = gf_v[pl.ds(j * _GCH + 16 * i, 16)]
            for t in range(16):
                r = 16 * i + t
                rb = (j * _GCH + r) * W
                m = gv[t] != 0.0
                for k in range(D // 16):
                    plsc.store_scatter(obuf_v, [rb + goffs[k]],
                                       jnp.where(m, g1[k], g0[k]))
                    plsc.store_scatter(obuf_v, [rb + poffs[k]],
                                       pbuf_v[j, r, pl.ds(16 * k, 16)])

        ow = pltpu.make_async_copy(
            obuf_v.at[pl.ds(j * _GCH * W, _GCH * W)],
            out_hbm.at[pl.ds(base * W + j * _GCH * W, _GCH * W)],
            osem)
        ow.start()
        owrites.append(ow)

    for ow in owrites:
        ow.wait()


@jax.jit
def kernel(scale, gender, age, perp, gender_table, perp_table):
    mesh = plsc.VectorSubcoreMesh(core_axis_name="c", subcore_axis_name="s")
    run = pl.kernel(
        _body,
        mesh=mesh,
        compiler_params=pltpu.CompilerParams(
            use_tc_tiling_on_sc=False, needs_layout_passes=False),
        out_type=jax.ShapeDtypeStruct((B * W,), jnp.float32),
        scratch_types=[
            pltpu.VMEM((_NG, _GCH), jnp.int32),
            pltpu.VMEM((_CPW,), jnp.float32),
            pltpu.VMEM((2, D), jnp.float32),
            pltpu.VMEM((_NG, _GCH, D), jnp.float32),
            pltpu.VMEM((2, _CPW), jnp.float32),
            pltpu.VMEM((_CPW * W,), jnp.float32),
            pltpu.SemaphoreType.DMA((_NG,)),
            pltpu.SemaphoreType.DMA,
        ],
    )
    flat = run(scale.reshape(_NW, _CPW),
               gender.reshape(_NW, _CPW).astype(jnp.float32),
               age.reshape(_NW, _CPW), perp.reshape(_NW, _NG, _GCH),
               gender_table, perp_table)
    return flat.reshape(B, W)
